# baseline (speedup 1.0000x reference)
import functools

import jax
import jax.numpy as jnp
from jax import lax
from jax.experimental import pallas as pl
from jax.experimental.pallas import tpu as pltpu
from jax.experimental.pallas import tpu_sc as plsc

N = 10000
NP = 10240
CH = 128
E = 320000
C = 128
NW = 32
NCHUNK = 80
EPT = NCHUNK * C
EPAD = EPT * NW
NACC = 10112
DUMMY = 10048
CPS = NCHUNK // 2
ROWS_PER_TILE = NP // 16
ACC_PER_TILE = NACC // 16
TCB = 512

_mesh = plsc.VectorSubcoreMesh(core_axis_name="c", subcore_axis_name="s")


@functools.partial(
    pl.kernel,
    mesh=_mesh,
    out_type=jax.ShapeDtypeStruct((2, NP, 16), jnp.float32),
    scratch_types=[
        pltpu.VMEM((NCHUNK, C), jnp.int32),
        pltpu.VMEM((C, 16), jnp.float32),
        pltpu.VMEM_SHARED((NP, 16), jnp.float32),
    ],
)
def _sc_degree(dst_hbm, deg_hbm, didx, vals, deg_sh):
    c = lax.axis_index("c")
    s = lax.axis_index("s")
    w = s * 2 + c

    @pl.loop(0, C)
    def _(r):
        vals[r, pl.ds(0, 16)] = jnp.zeros((16,), jnp.float32)

    base_row = s * ROWS_PER_TILE
    for k in range(ROWS_PER_TILE // C):
        pltpu.sync_copy(vals, deg_sh.at[pl.ds(base_row + k * C, C)])

    pltpu.sync_copy(dst_hbm.at[pl.ds(w * NCHUNK, NCHUNK)], didx)

    @pl.loop(0, C)
    def _(r):
        vals[r, pl.ds(0, 16)] = jnp.ones((16,), jnp.float32)

    plsc.subcore_barrier()

    @pl.loop(0, NCHUNK)
    def _(j):
        pltpu.sync_copy(vals, deg_sh.at[didx.at[j]], add=True)

    plsc.subcore_barrier()
    pltpu.sync_copy(
        deg_sh.at[pl.ds(base_row, ROWS_PER_TILE)],
        deg_hbm.at[c].at[pl.ds(base_row, ROWS_PER_TILE)],
    )


def _make_scatter(phase):
    base_chunk = phase * NW * CPS

    def body(hp_hbm, src_hbm, dst_hbm, acc_hbm,
             sidx, didx, rows0, rows1, acc_sh, sem0, sem1):
        c = lax.axis_index("c")
        s = lax.axis_index("s")
        w = s * 2 + c
        base_row = s * ACC_PER_TILE

        @pl.loop(0, C)
        def _(r):
            for k in range(CH // 16):
                rows0[r, pl.ds(k * 16, 16)] = jnp.zeros((16,), jnp.float32)

        for k in range(ACC_PER_TILE // C):
            pltpu.sync_copy(rows0, acc_sh.at[pl.ds(base_row + k * C, C)])
        pltpu.sync_copy(rows0.at[pl.ds(0, ACC_PER_TILE % C)],
                        acc_sh.at[pl.ds(base_row + (ACC_PER_TILE // C) * C,
                                        ACC_PER_TILE % C)])

        pltpu.sync_copy(src_hbm.at[pl.ds(base_chunk + w * CPS, CPS)], sidx)
        pltpu.sync_copy(dst_hbm.at[pl.ds(base_chunk + w * CPS, CPS)], didx)

        plsc.subcore_barrier()

        @pl.loop(0, CPS // 2)
        def _(jj):
            j = jj * 2
            cp0 = pltpu.async_copy(hp_hbm.at[sidx.at[j]], rows0, sem0)
            cp1 = pltpu.async_copy(hp_hbm.at[sidx.at[j + 1]], rows1, sem1)
            cp0.wait()
            pltpu.sync_copy(rows0, acc_sh.at[didx.at[j]], add=True)
            cp1.wait()
            pltpu.sync_copy(rows1, acc_sh.at[didx.at[j + 1]], add=True)

        plsc.subcore_barrier()
        pltpu.sync_copy(
            acc_sh.at[pl.ds(base_row, ACC_PER_TILE)],
            acc_hbm.at[c].at[pl.ds(base_row, ACC_PER_TILE)],
        )

    return pl.kernel(
        body,
        mesh=_mesh,
        out_type=jax.ShapeDtypeStruct((2, NP, CH), jnp.float32),
        scratch_types=[
            pltpu.VMEM((CPS, C), jnp.int32),
            pltpu.VMEM((CPS, C), jnp.int32),
            pltpu.VMEM((C, CH), jnp.float32),
            pltpu.VMEM((C, CH), jnp.float32),
            pltpu.VMEM_SHARED((NACC, CH), jnp.float32),
            pltpu.SemaphoreType.DMA,
            pltpu.SemaphoreType.DMA,
        ],
    )


_scatter_p0 = _make_scatter(0)
_scatter_p1 = _make_scatter(1)


def _mm_body(x_ref, w_ref, o_ref):
    o_ref[...] = jnp.dot(x_ref[...], w_ref[...],
                         preferred_element_type=jnp.float32)


def _tc_matmul(x, w):
    return pl.pallas_call(
        _mm_body,
        grid=(NP // TCB,),
        in_specs=[
            pl.BlockSpec((TCB, CH), lambda i: (i, 0)),
            pl.BlockSpec((CH, CH), lambda i: (0, 0)),
        ],
        out_specs=pl.BlockSpec((TCB, CH), lambda i: (i, 0)),
        out_shape=jax.ShapeDtypeStruct((NP, CH), jnp.float32),
    )(x, w)


def _prep_body(d0_ref, d1_ref, h_ref, dinv_ref, hp_ref):
    deg = d0_ref[...][:, 0:1] + d1_ref[...][:, 0:1] + 1.0
    db = jnp.broadcast_to(lax.rsqrt(deg), (TCB, CH))
    dinv_ref[...] = db
    hp_ref[...] = h_ref[...] * db


def _tc_prep(deg0, deg1, h1):
    return pl.pallas_call(
        _prep_body,
        grid=(NP // TCB,),
        in_specs=[
            pl.BlockSpec((TCB, 16), lambda i: (i, 0)),
            pl.BlockSpec((TCB, 16), lambda i: (i, 0)),
            pl.BlockSpec((TCB, CH), lambda i: (i, 0)),
        ],
        out_specs=[
            pl.BlockSpec((TCB, CH), lambda i: (i, 0)),
            pl.BlockSpec((TCB, CH), lambda i: (i, 0)),
        ],
        out_shape=[
            jax.ShapeDtypeStruct((NP, CH), jnp.float32),
            jax.ShapeDtypeStruct((NP, CH), jnp.float32),
        ],
    )(deg0, deg1, h1)


def _mid_body(a0_ref, a1_ref, a2_ref, a3_ref, hp_ref, db_ref, b_ref, w_ref,
              o_ref):
    db = db_ref[...]
    z = (a0_ref[...] + a1_ref[...] + a2_ref[...] + a3_ref[...]
         + hp_ref[...]) * db + b_ref[...]
    z = jnp.maximum(z, 0.0)
    o_ref[...] = jnp.dot(z, w_ref[...],
                         preferred_element_type=jnp.float32) * db


def _tc_mid(a0, a1, a2, a3, hp, db, b, w):
    return pl.pallas_call(
        _mid_body,
        grid=(NP // TCB,),
        in_specs=[
            pl.BlockSpec((TCB, CH), lambda i: (i, 0)),
            pl.BlockSpec((TCB, CH), lambda i: (i, 0)),
            pl.BlockSpec((TCB, CH), lambda i: (i, 0)),
            pl.BlockSpec((TCB, CH), lambda i: (i, 0)),
            pl.BlockSpec((TCB, CH), lambda i: (i, 0)),
            pl.BlockSpec((TCB, CH), lambda i: (i, 0)),
            pl.BlockSpec((1, CH), lambda i: (0, 0)),
            pl.BlockSpec((CH, CH), lambda i: (0, 0)),
        ],
        out_specs=pl.BlockSpec((TCB, CH), lambda i: (i, 0)),
        out_shape=jax.ShapeDtypeStruct((NP, CH), jnp.float32),
    )(a0, a1, a2, a3, hp, db, b, w)


def _fin_body(a0_ref, a1_ref, a2_ref, a3_ref, hp_ref, db_ref, b_ref, o_ref):
    o_ref[...] = ((a0_ref[...] + a1_ref[...] + a2_ref[...] + a3_ref[...]
                   + hp_ref[...]) * db_ref[...] + b_ref[...])


def _tc_final(a0, a1, a2, a3, hp, db, b):
    return pl.pallas_call(
        _fin_body,
        grid=(NP // TCB,),
        in_specs=[
            pl.BlockSpec((TCB, CH), lambda i: (i, 0)),
            pl.BlockSpec((TCB, CH), lambda i: (i, 0)),
            pl.BlockSpec((TCB, CH), lambda i: (i, 0)),
            pl.BlockSpec((TCB, CH), lambda i: (i, 0)),
            pl.BlockSpec((TCB, CH), lambda i: (i, 0)),
            pl.BlockSpec((TCB, CH), lambda i: (i, 0)),
            pl.BlockSpec((1, CH), lambda i: (0, 0)),
        ],
        out_specs=pl.BlockSpec((TCB, CH), lambda i: (i, 0)),
        out_shape=jax.ShapeDtypeStruct((NP, CH), jnp.float32),
    )(a0, a1, a2, a3, hp, db, b)


@jax.jit
def kernel(x, edge_index, W1, b1, W2, b2):
    src = edge_index[0].astype(jnp.int32)
    dst = edge_index[1].astype(jnp.int32)
    pad = EPAD - E
    src2d = jnp.concatenate(
        [src, jnp.zeros((pad,), jnp.int32)]).reshape(EPAD // C, C)
    dst2d = jnp.concatenate(
        [dst, jnp.full((pad,), DUMMY, jnp.int32)]).reshape(EPAD // C, C)
    x_p = jnp.pad(x, ((0, NP - N), (0, 0)))
    b1r = b1.reshape(1, CH)
    b2r = b2.reshape(1, CH)

    h1 = _tc_matmul(x_p, W1)
    deg = _sc_degree(dst2d)
    db, hp1 = _tc_prep(deg[0], deg[1], h1)

    accA = _scatter_p0(hp1, src2d, dst2d)
    accB = _scatter_p1(hp1, src2d, dst2d)
    hp2 = _tc_mid(accA[0], accA[1], accB[0], accB[1], hp1, db, b1r, W2)

    accC = _scatter_p0(hp2, src2d, dst2d)
    accD = _scatter_p1(hp2, src2d, dst2d)
    out = _tc_final(accC[0], accC[1], accD[0], accD[1], hp2, db, b2r)
    return out[:N]


# speedup vs baseline: 2.2805x; 2.2805x over previous
import functools

import jax
import jax.numpy as jnp
from jax import lax
from jax.experimental import pallas as pl
from jax.experimental.pallas import tpu as pltpu
from jax.experimental.pallas import tpu_sc as plsc

N = 10000
NP = 10240
CH = 128
E = 320000
C = 128
NW = 32
NCHUNK = 80
EPT = NCHUNK * C
EPAD = EPT * NW
NACC = 10112
CPS = NCHUNK // 2
ROWS_PER_TILE = NP // 16
ACC_PER_TILE = NACC // 16
TCB = 512

_mesh = plsc.VectorSubcoreMesh(core_axis_name="c", subcore_axis_name="s")


@functools.partial(
    pl.kernel,
    mesh=_mesh,
    out_type=jax.ShapeDtypeStruct((2, NP, 16), jnp.float32),
    scratch_types=[
        pltpu.VMEM((NCHUNK, C), jnp.int32),
        pltpu.VMEM((C, 16), jnp.float32),
        pltpu.VMEM_SHARED((NP, 16), jnp.float32),
    ],
)
def _sc_degree(dst_hbm, deg_hbm, didx, vals, deg_sh):
    c = lax.axis_index("c")
    s = lax.axis_index("s")
    w = s * 2 + c

    @pl.loop(0, C)
    def _(r):
        vals[r, pl.ds(0, 16)] = jnp.zeros((16,), jnp.float32)

    base_row = s * ROWS_PER_TILE
    for k in range(ROWS_PER_TILE // C):
        pltpu.sync_copy(vals, deg_sh.at[pl.ds(base_row + k * C, C)])

    pltpu.sync_copy(dst_hbm.at[pl.ds(w * NCHUNK, NCHUNK)], didx)

    @pl.loop(0, C)
    def _(r):
        vals[r, pl.ds(0, 16)] = jnp.ones((16,), jnp.float32)

    plsc.subcore_barrier()

    @pl.loop(0, NCHUNK)
    def _(j):
        pltpu.sync_copy(vals, deg_sh.at[didx.at[j]], add=True)

    plsc.subcore_barrier()
    pltpu.sync_copy(
        deg_sh.at[pl.ds(base_row, ROWS_PER_TILE)],
        deg_hbm.at[c].at[pl.ds(base_row, ROWS_PER_TILE)],
    )


def _make_scatter(phase):
    base_chunk = phase * NW * CPS

    def body(hp_hbm, src_hbm, dst_hbm, acc_hbm,
             sidx, didx, rows0, rows1, acc_sh, sem0, sem1):
        c = lax.axis_index("c")
        s = lax.axis_index("s")
        w = s * 2 + c
        base_row = s * ACC_PER_TILE

        @pl.loop(0, C)
        def _(r):
            for k in range(CH // 16):
                rows0[r, pl.ds(k * 16, 16)] = jnp.zeros((16,), jnp.float32)

        for k in range(ACC_PER_TILE // C):
            pltpu.sync_copy(rows0, acc_sh.at[pl.ds(base_row + k * C, C)])
        pltpu.sync_copy(rows0.at[pl.ds(0, ACC_PER_TILE % C)],
                        acc_sh.at[pl.ds(base_row + (ACC_PER_TILE // C) * C,
                                        ACC_PER_TILE % C)])

        pltpu.sync_copy(src_hbm.at[pl.ds(base_chunk + w * CPS, CPS)], sidx)
        pltpu.sync_copy(dst_hbm.at[pl.ds(base_chunk + w * CPS, CPS)], didx)

        plsc.subcore_barrier()

        @pl.loop(0, CPS // 2)
        def _(jj):
            j = jj * 2
            cp0 = pltpu.async_copy(hp_hbm.at[sidx.at[j]], rows0, sem0)
            cp1 = pltpu.async_copy(hp_hbm.at[sidx.at[j + 1]], rows1, sem1)
            cp0.wait()
            pltpu.sync_copy(rows0, acc_sh.at[didx.at[j]], add=True)
            cp1.wait()
            pltpu.sync_copy(rows1, acc_sh.at[didx.at[j + 1]], add=True)

        plsc.subcore_barrier()
        pltpu.sync_copy(
            acc_sh.at[pl.ds(base_row, ACC_PER_TILE)],
            acc_hbm.at[c].at[pl.ds(base_row, ACC_PER_TILE)],
        )

    return pl.kernel(
        body,
        mesh=_mesh,
        out_type=jax.ShapeDtypeStruct((2, NP, CH), jnp.float32),
        scratch_types=[
            pltpu.VMEM((CPS, C), jnp.int32),
            pltpu.VMEM((CPS, C), jnp.int32),
            pltpu.VMEM((C, CH), jnp.float32),
            pltpu.VMEM((C, CH), jnp.float32),
            pltpu.VMEM_SHARED((NACC, CH), jnp.float32),
            pltpu.SemaphoreType.DMA,
            pltpu.SemaphoreType.DMA,
        ],
    )


_scatter_p0 = _make_scatter(0)
_scatter_p1 = _make_scatter(1)


def _mm_body(x_ref, w_ref, o_ref):
    o_ref[...] = jnp.dot(x_ref[...], w_ref[...],
                         preferred_element_type=jnp.float32)


def _tc_matmul(x, w):
    return pl.pallas_call(
        _mm_body,
        grid=(NP // TCB,),
        in_specs=[
            pl.BlockSpec((TCB, CH), lambda i: (i, 0)),
            pl.BlockSpec((CH, CH), lambda i: (0, 0)),
        ],
        out_specs=pl.BlockSpec((TCB, CH), lambda i: (i, 0)),
        out_shape=jax.ShapeDtypeStruct((NP, CH), jnp.float32),
    )(x, w)


def _prep_body(d0_ref, d1_ref, h_ref, dinv_ref, hp_ref):
    deg = d0_ref[...][:, 0:1] + d1_ref[...][:, 0:1] + 1.0
    db = jnp.broadcast_to(lax.rsqrt(deg), (TCB, CH))
    dinv_ref[...] = db
    hp_ref[...] = h_ref[...] * db


def _tc_prep(deg0, deg1, h1):
    return pl.pallas_call(
        _prep_body,
        grid=(NP // TCB,),
        in_specs=[
            pl.BlockSpec((TCB, 16), lambda i: (i, 0)),
            pl.BlockSpec((TCB, 16), lambda i: (i, 0)),
            pl.BlockSpec((TCB, CH), lambda i: (i, 0)),
        ],
        out_specs=[
            pl.BlockSpec((TCB, CH), lambda i: (i, 0)),
            pl.BlockSpec((TCB, CH), lambda i: (i, 0)),
        ],
        out_shape=[
            jax.ShapeDtypeStruct((NP, CH), jnp.float32),
            jax.ShapeDtypeStruct((NP, CH), jnp.float32),
        ],
    )(deg0, deg1, h1)


def _mid_body(a0_ref, a1_ref, a2_ref, a3_ref, hp_ref, db_ref, b_ref, w_ref,
              o_ref):
    db = db_ref[...]
    z = (a0_ref[...] + a1_ref[...] + a2_ref[...] + a3_ref[...]
         + hp_ref[...]) * db + b_ref[...]
    z = jnp.maximum(z, 0.0)
    o_ref[...] = jnp.dot(z, w_ref[...],
                         preferred_element_type=jnp.float32) * db


def _tc_mid(a0, a1, a2, a3, hp, db, b, w):
    return pl.pallas_call(
        _mid_body,
        grid=(NP // TCB,),
        in_specs=[
            pl.BlockSpec((TCB, CH), lambda i: (i, 0)),
            pl.BlockSpec((TCB, CH), lambda i: (i, 0)),
            pl.BlockSpec((TCB, CH), lambda i: (i, 0)),
            pl.BlockSpec((TCB, CH), lambda i: (i, 0)),
            pl.BlockSpec((TCB, CH), lambda i: (i, 0)),
            pl.BlockSpec((TCB, CH), lambda i: (i, 0)),
            pl.BlockSpec((1, CH), lambda i: (0, 0)),
            pl.BlockSpec((CH, CH), lambda i: (0, 0)),
        ],
        out_specs=pl.BlockSpec((TCB, CH), lambda i: (i, 0)),
        out_shape=jax.ShapeDtypeStruct((NP, CH), jnp.float32),
    )(a0, a1, a2, a3, hp, db, b, w)


def _fin_body(a0_ref, a1_ref, a2_ref, a3_ref, hp_ref, db_ref, b_ref, o_ref):
    o_ref[...] = ((a0_ref[...] + a1_ref[...] + a2_ref[...] + a3_ref[...]
                   + hp_ref[...]) * db_ref[...] + b_ref[...])


def _tc_final(a0, a1, a2, a3, hp, db, b):
    return pl.pallas_call(
        _fin_body,
        grid=(NP // TCB,),
        in_specs=[
            pl.BlockSpec((TCB, CH), lambda i: (i, 0)),
            pl.BlockSpec((TCB, CH), lambda i: (i, 0)),
            pl.BlockSpec((TCB, CH), lambda i: (i, 0)),
            pl.BlockSpec((TCB, CH), lambda i: (i, 0)),
            pl.BlockSpec((TCB, CH), lambda i: (i, 0)),
            pl.BlockSpec((TCB, CH), lambda i: (i, 0)),
            pl.BlockSpec((1, CH), lambda i: (0, 0)),
        ],
        out_specs=pl.BlockSpec((TCB, CH), lambda i: (i, 0)),
        out_shape=jax.ShapeDtypeStruct((NP, CH), jnp.float32),
    )(a0, a1, a2, a3, hp, db, b)


@jax.jit
def kernel(x, edge_index, W1, b1, W2, b2):
    src = edge_index[0].astype(jnp.int32)
    dst = edge_index[1].astype(jnp.int32)
    pad = EPAD - E
    pad_src = jnp.arange(pad, dtype=jnp.int32) % N
    pad_dst = N + jnp.arange(pad, dtype=jnp.int32) % (NACC - N)
    src2d = jnp.concatenate([src, pad_src]).reshape(EPAD // C, C)
    dst2d = jnp.concatenate([dst, pad_dst]).reshape(EPAD // C, C)
    x_p = jnp.pad(x, ((0, NP - N), (0, 0)))
    b1r = b1.reshape(1, CH)
    b2r = b2.reshape(1, CH)

    h1 = _tc_matmul(x_p, W1)
    deg = _sc_degree(dst2d)
    db, hp1 = _tc_prep(deg[0], deg[1], h1)

    accA = _scatter_p0(hp1, src2d, dst2d)
    accB = _scatter_p1(hp1, src2d, dst2d)
    hp2 = _tc_mid(accA[0], accA[1], accB[0], accB[1], hp1, db, b1r, W2)

    accC = _scatter_p0(hp2, src2d, dst2d)
    accD = _scatter_p1(hp2, src2d, dst2d)
    out = _tc_final(accC[0], accC[1], accD[0], accD[1], hp2, db, b2r)
    return out[:N]
